# baseline (speedup 1.0000x reference)
import jax
import jax.numpy as jnp
from jax.experimental import pallas as pl
from jax.experimental.pallas import tpu as pltpu

B, T, F, E, H, HG, C = 4, 2048, 1024, 8, 2048, 64, 1000
TOPK = 2
NP = B * TOPK
TTG = 512
NTG = T // TTG
HT = 512
NH = H // HT
LG = 128

_SQRT2 = 1.4142135623730951


def _gelu(v):
    return 0.5 * v * (1.0 + jax.lax.erf(v / _SQRT2))


def _gating_kernel(x_ref, wg1_ref, bg1_ref, lng_ref, lnb_ref, wg2_ref,
                   bg2_ref, xb_ref, w_out_ref, i_out_ref, acc_ref):
    t = pl.program_id(0)

    @pl.when(t == 0)
    def _():
        acc_ref[...] = jnp.zeros_like(acc_ref)

    xt = x_ref[...]
    xb_ref[...] = xt.astype(jnp.bfloat16)
    acc_ref[0:B, :] += jnp.sum(xt, axis=1)

    @pl.when(t == NTG - 1)
    def _():
        g = acc_ref[0:B, :] / T
        h = jnp.dot(g, wg1_ref[...], preferred_element_type=jnp.float32)
        h = h + bg1_ref[...]
        h = _gelu(h)
        col = jax.lax.broadcasted_iota(jnp.int32, (B, LG), 1)
        row = jax.lax.broadcasted_iota(jnp.int32, (B, LG), 0)
        real = col < HG
        mu = jnp.sum(h, axis=-1, keepdims=True) / HG
        d = jnp.where(real, h - mu, 0.0)
        var = jnp.sum(d * d, axis=-1, keepdims=True) / HG
        hn = (h - mu) / jnp.sqrt(var + 1e-5) * lng_ref[...] + lnb_ref[...]
        logits = jnp.dot(hn, wg2_ref[...], preferred_element_type=jnp.float32)
        logits = logits + bg2_ref[...]
        logits = jnp.where(col < E, logits, -1e30)
        m = jnp.max(logits, axis=-1, keepdims=True)
        ex = jnp.exp(logits - m)
        rw = ex / jnp.sum(ex, axis=-1, keepdims=True)
        v1 = jnp.max(rw, axis=-1, keepdims=True)
        i1 = jnp.min(jnp.where(rw == v1, col, LG), axis=-1, keepdims=True)
        rw2 = jnp.where(col == i1, -1.0, rw)
        v2 = jnp.max(rw2, axis=-1, keepdims=True)
        i2 = jnp.min(jnp.where(rw2 == v2, col, LG), axis=-1, keepdims=True)
        s = v1 + v2 + 1e-8
        w1 = v1 / s
        w2 = v2 / s
        wmat = jnp.where(col == 0, w1, jnp.where(col == 1, w2, 0.0))
        emat = jnp.where(col == 0, i1, jnp.where(col == 1, i2, 0))
        slot = 2 * row + col
        keymat = jnp.where(col < TOPK, emat * 16 + slot, 99999)
        colr = jax.lax.broadcasted_iota(jnp.int32, (1, LG), 1)
        es_row = jnp.zeros((1, LG), jnp.int32)
        ob_row = jnp.zeros((1, LG), jnp.int32)
        ws_row = jnp.zeros((1, LG), jnp.float32)
        for p in range(NP):
            mk = jnp.min(keymat)
            es_p = mk // 16
            ob_p = (mk - es_p * 16) // 2
            w_p = jnp.sum(jnp.where(keymat == mk, wmat, 0.0))
            sel = colr == p
            es_row = jnp.where(sel, es_p, es_row)
            ob_row = jnp.where(sel, ob_p, ob_row)
            ws_row = jnp.where(sel, w_p, ws_row)
            keymat = jnp.where(keymat == mk, 99999, keymat)
        w_out_ref[...] = jnp.zeros_like(w_out_ref)
        i_out_ref[...] = jnp.zeros_like(i_out_ref)
        w_out_ref[0:1, :] = ws_row
        i_out_ref[0:1, :] = es_row
        i_out_ref[1:2, :] = ob_row


def _expert_kernel(es_ref, ob_ref, ws_ref, x_ref, w1_ref, b1_ref,
                   w2_ref, b2_ref, out_ref, acc_ref):
    ht = pl.program_id(0)
    p = pl.program_id(1)

    @pl.when(jnp.logical_and(ht == 0, p == 0))
    def _():
        acc_ref[...] = jnp.zeros_like(acc_ref)

    ob = ob_ref[p]
    w = ws_ref[p]
    xr = x_ref[ob]

    h = jnp.dot(xr, w1_ref[0].astype(jnp.bfloat16),
                preferred_element_type=jnp.float32)
    h = _gelu(h + b1_ref[0])
    ones = jnp.full((1, T), 1.0, jnp.bfloat16)
    pe = jnp.dot(ones, h.astype(jnp.bfloat16),
                 preferred_element_type=jnp.float32) / T
    part = jnp.dot(pe.astype(jnp.bfloat16), w2_ref[0].astype(jnp.bfloat16),
                   preferred_element_type=jnp.float32)
    contrib = w * part

    @pl.when(ht == 0)
    def _():
        acc_ref[pl.ds(ob, 1), :] += contrib + w * b2_ref[0]

    @pl.when(ht != 0)
    def _():
        acc_ref[pl.ds(ob, 1), :] += contrib

    @pl.when(jnp.logical_and(ht == NH - 1, p == NP - 1))
    def _():
        out_ref[...] = acc_ref[0:B, :][:, None, :]


def kernel(x, Wg1, bg1, ln_g, ln_b, Wg2, bg2, W1, b1, W2, b2):
    f32 = jnp.float32
    Wg1p = jnp.pad(Wg1, ((0, 0), (0, LG - HG)))
    bg1p = jnp.pad(bg1, (0, LG - HG)).reshape(1, LG)
    lngp = jnp.pad(ln_g, (0, LG - HG)).reshape(1, LG)
    lnbp = jnp.pad(ln_b, (0, LG - HG)).reshape(1, LG)
    Wg2p = jnp.pad(Wg2, ((0, LG - HG), (0, LG - E)))
    bg2p = jnp.pad(bg2, (0, LG - E)).reshape(1, LG)

    xb, w_out, i_out = pl.pallas_call(
        _gating_kernel,
        grid=(NTG,),
        in_specs=[
            pl.BlockSpec((B, TTG, F), lambda t: (0, t, 0)),
            pl.BlockSpec((F, LG), lambda t: (0, 0)),
            pl.BlockSpec((1, LG), lambda t: (0, 0)),
            pl.BlockSpec((1, LG), lambda t: (0, 0)),
            pl.BlockSpec((1, LG), lambda t: (0, 0)),
            pl.BlockSpec((LG, LG), lambda t: (0, 0)),
            pl.BlockSpec((1, LG), lambda t: (0, 0)),
        ],
        out_specs=[
            pl.BlockSpec((B, TTG, F), lambda t: (0, t, 0)),
            pl.BlockSpec((8, LG), lambda t: (0, 0)),
            pl.BlockSpec((8, LG), lambda t: (0, 0)),
        ],
        out_shape=[
            jax.ShapeDtypeStruct((B, T, F), jnp.bfloat16),
            jax.ShapeDtypeStruct((8, LG), f32),
            jax.ShapeDtypeStruct((8, LG), jnp.int32),
        ],
        scratch_shapes=[pltpu.VMEM((8, F), f32)],
    )(x, Wg1p, bg1p, lngp, lnbp, Wg2p, bg2p)

    ws = w_out[0, :NP]
    es = i_out[0, :NP]
    ob = i_out[1, :NP]

    b1r = b1.reshape(E, 1, H)
    b2r = b2.reshape(E, 1, C)

    grid_spec = pltpu.PrefetchScalarGridSpec(
        num_scalar_prefetch=3,
        grid=(NH, NP),
        in_specs=[
            pl.BlockSpec((B, T, F), lambda ht, p, es, ob, ws: (0, 0, 0)),
            pl.BlockSpec((1, F, HT), lambda ht, p, es, ob, ws:
                         (es[p], 0, ht)),
            pl.BlockSpec((1, 1, HT), lambda ht, p, es, ob, ws:
                         (es[p], 0, ht)),
            pl.BlockSpec((1, HT, C), lambda ht, p, es, ob, ws:
                         (es[p], ht, 0)),
            pl.BlockSpec((1, 1, C), lambda ht, p, es, ob, ws:
                         (es[p], 0, 0)),
        ],
        out_specs=pl.BlockSpec((B, 1, C), lambda ht, p, es, ob, ws:
                               (0, 0, 0)),
        scratch_shapes=[pltpu.VMEM((8, C), f32)],
    )

    out = pl.pallas_call(
        _expert_kernel,
        grid_spec=grid_spec,
        out_shape=jax.ShapeDtypeStruct((B, 1, C), f32),
        compiler_params=pltpu.CompilerParams(
            dimension_semantics=("arbitrary", "arbitrary")),
    )(es, ob, ws, xb, W1, b1r, W2, b2r)

    return out.reshape(B, C)


# speedup vs baseline: 1.4624x; 1.4624x over previous
import jax
import jax.numpy as jnp
from jax.experimental import pallas as pl
from jax.experimental.pallas import tpu as pltpu

B, T, F, E, H, HG, C = 4, 2048, 1024, 8, 2048, 64, 1000
TOPK = 2
NP = B * TOPK
TTG = 512
NTG = T // TTG
HT = 512
NH = H // HT
LG = 128

_SQRT2 = 1.4142135623730951


def _gelu(v):
    return 0.5 * v * (1.0 + jax.lax.erf(v / _SQRT2))


def _gating_kernel(x_ref, wg1_ref, bg1_ref, lng_ref, lnb_ref, wg2_ref,
                   bg2_ref, xb_ref, w_out_ref, i_out_ref, acc_ref):
    t = pl.program_id(0)

    @pl.when(t == 0)
    def _():
        acc_ref[...] = jnp.zeros_like(acc_ref)

    xt = x_ref[...]
    xb_ref[...] = xt.astype(jnp.bfloat16)
    acc_ref[0:B, :] += jnp.sum(xt, axis=1)

    @pl.when(t == NTG - 1)
    def _():
        g = acc_ref[0:B, :] / T
        h = jnp.dot(g, wg1_ref[...], preferred_element_type=jnp.float32)
        h = h + bg1_ref[...]
        h = _gelu(h)
        col = jax.lax.broadcasted_iota(jnp.int32, (B, LG), 1)
        real = col < HG
        mu = jnp.sum(h, axis=-1, keepdims=True) / HG
        d = jnp.where(real, h - mu, 0.0)
        var = jnp.sum(d * d, axis=-1, keepdims=True) / HG
        hn = (h - mu) / jnp.sqrt(var + 1e-5) * lng_ref[...] + lnb_ref[...]
        logits = jnp.dot(hn, wg2_ref[...], preferred_element_type=jnp.float32)
        logits = logits + bg2_ref[...]
        logits = jnp.where(col < E, logits, -1e30)
        m = jnp.max(logits, axis=-1, keepdims=True)
        ex = jnp.exp(logits - m)
        rw = ex / jnp.sum(ex, axis=-1, keepdims=True)
        v1 = jnp.max(rw, axis=-1, keepdims=True)
        i1 = jnp.min(jnp.where(rw == v1, col, LG), axis=-1, keepdims=True)
        rw2 = jnp.where(col == i1, -1.0, rw)
        v2 = jnp.max(rw2, axis=-1, keepdims=True)
        i2 = jnp.min(jnp.where(rw2 == v2, col, LG), axis=-1, keepdims=True)
        s = v1 + v2 + 1e-8
        w1 = v1 / s
        w2 = v2 / s
        w_out_ref[...] = jnp.zeros_like(w_out_ref)
        i_out_ref[...] = jnp.zeros_like(i_out_ref)
        w_out_ref[0:B, :] = jnp.where(col == 0, w1,
                                      jnp.where(col == 1, w2, 0.0))
        i_out_ref[0:B, :] = jnp.where(col == 0, i1,
                                      jnp.where(col == 1, i2, 0))


def _expert_kernel(eidx_ref, wts_ref, x_ref, w1a_ref, w1b_ref, b1a_ref,
                   b1b_ref, w2a_ref, w2b_ref, b2a_ref, b2b_ref, out_ref):
    b = pl.program_id(0)
    ht = pl.program_id(1)
    xr = x_ref[b]
    wa = wts_ref[TOPK * b]
    wb = wts_ref[TOPK * b + 1]
    ones = jnp.full((1, T), 1.0, jnp.bfloat16)
    cdims = (((1,), (1,)), ((), ()))

    ha = jnp.dot(xr, w1a_ref[0].astype(jnp.bfloat16),
                 preferred_element_type=jnp.float32)
    hb = jnp.dot(xr, w1b_ref[0].astype(jnp.bfloat16),
                 preferred_element_type=jnp.float32)
    ha = _gelu(ha + b1a_ref[0])
    hb = _gelu(hb + b1b_ref[0])
    pea = jnp.dot(ones, ha.astype(jnp.bfloat16),
                  preferred_element_type=jnp.float32) / T
    peb = jnp.dot(ones, hb.astype(jnp.bfloat16),
                  preferred_element_type=jnp.float32) / T
    parta = jax.lax.dot_general(pea.astype(jnp.bfloat16),
                                w2a_ref[0].astype(jnp.bfloat16),
                                cdims, preferred_element_type=jnp.float32)
    partb = jax.lax.dot_general(peb.astype(jnp.bfloat16),
                                w2b_ref[0].astype(jnp.bfloat16),
                                cdims, preferred_element_type=jnp.float32)
    contrib = wa * parta + wb * partb

    @pl.when(ht == 0)
    def _():
        out_ref[0] = contrib + wa * b2a_ref[0] + wb * b2b_ref[0]

    @pl.when(ht != 0)
    def _():
        out_ref[0] += contrib


def kernel(x, Wg1, bg1, ln_g, ln_b, Wg2, bg2, W1, b1, W2, b2):
    f32 = jnp.float32
    Wg1p = jnp.pad(Wg1, ((0, 0), (0, LG - HG)))
    bg1p = jnp.pad(bg1, (0, LG - HG)).reshape(1, LG)
    lngp = jnp.pad(ln_g, (0, LG - HG)).reshape(1, LG)
    lnbp = jnp.pad(ln_b, (0, LG - HG)).reshape(1, LG)
    Wg2p = jnp.pad(Wg2, ((0, LG - HG), (0, LG - E)))
    bg2p = jnp.pad(bg2, (0, LG - E)).reshape(1, LG)

    xb, w_out, i_out = pl.pallas_call(
        _gating_kernel,
        grid=(NTG,),
        in_specs=[
            pl.BlockSpec((B, TTG, F), lambda t: (0, t, 0)),
            pl.BlockSpec((F, LG), lambda t: (0, 0)),
            pl.BlockSpec((1, LG), lambda t: (0, 0)),
            pl.BlockSpec((1, LG), lambda t: (0, 0)),
            pl.BlockSpec((1, LG), lambda t: (0, 0)),
            pl.BlockSpec((LG, LG), lambda t: (0, 0)),
            pl.BlockSpec((1, LG), lambda t: (0, 0)),
        ],
        out_specs=[
            pl.BlockSpec((B, TTG, F), lambda t: (0, t, 0)),
            pl.BlockSpec((8, LG), lambda t: (0, 0)),
            pl.BlockSpec((8, LG), lambda t: (0, 0)),
        ],
        out_shape=[
            jax.ShapeDtypeStruct((B, T, F), jnp.bfloat16),
            jax.ShapeDtypeStruct((8, LG), f32),
            jax.ShapeDtypeStruct((8, LG), jnp.int32),
        ],
        scratch_shapes=[pltpu.VMEM((8, F), f32)],
    )(x, Wg1p, bg1p, lngp, lnbp, Wg2p, bg2p)

    wflat = w_out[:B, :TOPK].reshape(NP)
    eflat = i_out[:B, :TOPK].reshape(NP)

    b1r = b1.reshape(E, 1, H)
    b2r = b2.reshape(E, 1, C)
    W2t = jnp.swapaxes(W2, 1, 2)

    grid_spec = pltpu.PrefetchScalarGridSpec(
        num_scalar_prefetch=2,
        grid=(B, NH),
        in_specs=[
            pl.BlockSpec((B, T, F), lambda b, ht, eidx, wts: (0, 0, 0)),
            pl.BlockSpec((1, F, HT), lambda b, ht, eidx, wts:
                         (eidx[TOPK * b], 0, ht)),
            pl.BlockSpec((1, F, HT), lambda b, ht, eidx, wts:
                         (eidx[TOPK * b + 1], 0, ht)),
            pl.BlockSpec((1, 1, HT), lambda b, ht, eidx, wts:
                         (eidx[TOPK * b], 0, ht)),
            pl.BlockSpec((1, 1, HT), lambda b, ht, eidx, wts:
                         (eidx[TOPK * b + 1], 0, ht)),
            pl.BlockSpec((1, C, HT), lambda b, ht, eidx, wts:
                         (eidx[TOPK * b], 0, ht)),
            pl.BlockSpec((1, C, HT), lambda b, ht, eidx, wts:
                         (eidx[TOPK * b + 1], 0, ht)),
            pl.BlockSpec((1, 1, C), lambda b, ht, eidx, wts:
                         (eidx[TOPK * b], 0, 0)),
            pl.BlockSpec((1, 1, C), lambda b, ht, eidx, wts:
                         (eidx[TOPK * b + 1], 0, 0)),
        ],
        out_specs=pl.BlockSpec((1, 1, C), lambda b, ht, eidx, wts:
                               (b, 0, 0)),
    )

    out = pl.pallas_call(
        _expert_kernel,
        grid_spec=grid_spec,
        out_shape=jax.ShapeDtypeStruct((B, 1, C), f32),
        compiler_params=pltpu.CompilerParams(
            dimension_semantics=("arbitrary", "arbitrary")),
    )(eflat, wflat, xb, W1, W1, b1r, b1r, W2t, W2t, b2r, b2r)

    return out.reshape(B, C)
